# baseline (speedup 1.0000x reference)
import functools

import jax
import jax.numpy as jnp
from jax import lax
from jax.experimental import pallas as pl
from jax.experimental.pallas import tpu as pltpu
from jax.experimental.pallas import tpu_sc as plsc

_NC = 2
_NS = 16
_NW = _NC * _NS
_L = 16


def _sc_partials(x0, x1, batch, row_base, n_rows, num_segments, chunk):
    G = num_segments
    R = n_rows // _NW
    CHUNK = chunk
    NCHUNK = R // CHUNK
    BG = 5
    BLOCK = BG * _L
    NBLOCK = CHUNK // BLOCK
    NBUF = 2

    mesh = plsc.VectorSubcoreMesh(core_axis_name="c", subcore_axis_name="s")

    @functools.partial(
        pl.kernel,
        out_type=jax.ShapeDtypeStruct((_NW, 3 * G), jnp.float32),
        mesh=mesh,
        scratch_types=[
            pltpu.VMEM((CHUNK,), jnp.float32),
            pltpu.VMEM((CHUNK,), jnp.float32),
            pltpu.VMEM((CHUNK,), jnp.float32),
            pltpu.VMEM((CHUNK,), jnp.float32),
            pltpu.VMEM((CHUNK,), jnp.int32),
            pltpu.VMEM((CHUNK,), jnp.int32),
            pltpu.VMEM((3 * G,), jnp.float32),
            pltpu.SemaphoreType.DMA,
            pltpu.SemaphoreType.DMA,
        ],
        compiler_params=pltpu.CompilerParams(needs_layout_passes=False),
    )
    def k(x0_hbm, x1_hbm, b_hbm, out_hbm, x0b0, x0b1, x1b0, x1b1,
          bbf0, bbf1, acc, sem0, sem1):
        wid = lax.axis_index("s") * _NC + lax.axis_index("c")
        base = wid * R
        sems = (sem0, sem1)
        x0b = (x0b0, x0b1)
        x1b = (x1b0, x1b1)
        bbf = (bbf0, bbf1)

        zeros_f = jnp.zeros((_L,), jnp.float32)

        def zero_body(i, _):
            acc[pl.ds(i * _L, _L)] = zeros_f
            return 0

        lax.fori_loop(0, (3 * G) // _L, zero_body, 0)

        ones_f = jnp.ones((_L,), jnp.float32)
        zeros_v = jnp.zeros((_L,), jnp.float32)
        iota = lax.iota(jnp.int32, _L)
        lane3 = iota < 3
        giota = G * iota

        def flush(run_id, cnt, v0, v1):
            sa = jnp.sum(v0)
            sb = jnp.sum(v1)
            val = jnp.where(iota == 0, sa, jnp.where(iota == 1, sb, cnt))
            plsc.addupdate_scatter(acc, [run_id + giota], val, mask=lane3)

        def start(ci, slot):
            off = base + ci * CHUNK
            return (
                pltpu.async_copy(x0_hbm.at[pl.ds(off, CHUNK)],
                                 x0b[slot], sems[slot]),
                pltpu.async_copy(x1_hbm.at[pl.ds(off, CHUNK)],
                                 x1b[slot], sems[slot]),
                pltpu.async_copy(b_hbm.at[pl.ds(row_base + off, CHUNK)],
                                 bbf[slot], sems[slot]),
            )

        def group_step(slot, gb, carry):
            run_id, cnt, v0, v1 = carry
            bb = bbf[slot][pl.ds(gb, _L)]
            id_hi = bb[_L - 1]
            xa = x0b[slot][pl.ds(gb, _L)]
            xb = x1b[slot][pl.ds(gb, _L)]

            def cont(run_id, cnt, v0, v1):
                return run_id, cnt + 16.0, v0 + xa, v1 + xb

            def brk(run_id, cnt, v0, v1):
                flush(run_id, cnt, v0, v1)
                id_lo = bb[0]

                def uni(_):
                    return id_hi, 16.0, xa, xb

                def mixed(_):
                    plsc.addupdate_scatter(acc, [bb], xa)
                    plsc.addupdate_scatter(acc, [bb + G], xb)
                    plsc.addupdate_scatter(acc, [bb + 2 * G], ones_f)
                    return id_hi, 0.0, zeros_v, zeros_v

                return lax.cond(id_lo == id_hi, uni, mixed, 0)

            return lax.cond(id_hi == run_id, cont, brk, run_id, cnt, v0, v1)

        def process(slot, carry):
            def blk(i, carry):
                run_id, cnt, v0, v1 = carry
                gb = i * BLOCK
                btail = bbf[slot][pl.ds(gb + BLOCK - _L, _L)]
                id_hi = btail[_L - 1]

                def fast(run_id, cnt, v0, v1):
                    xs0 = [x0b[slot][pl.ds(gb + j * _L, _L)]
                           for j in range(BG)]
                    xs1 = [x1b[slot][pl.ds(gb + j * _L, _L)]
                           for j in range(BG)]
                    s0 = (xs0[0] + xs0[1]) + (xs0[2] + xs0[3]) + xs0[4]
                    s1 = (xs1[0] + xs1[1]) + (xs1[2] + xs1[3]) + xs1[4]
                    return run_id, cnt + float(BLOCK), v0 + s0, v1 + s1

                def slow(run_id, cnt, v0, v1):
                    carry2 = (run_id, cnt, v0, v1)
                    for j in range(BG):
                        carry2 = group_step(slot, gb + j * _L, carry2)
                    return carry2

                return lax.cond(id_hi == run_id, fast, slow,
                                run_id, cnt, v0, v1)

            return lax.fori_loop(0, NBLOCK, blk, carry)

        hs = start(0, 0)
        carry = (jnp.int32(0), jnp.float32(0.0), zeros_v, zeros_v)
        for ci in range(NCHUNK):
            slot = ci % NBUF
            for h in hs:
                h.wait()
            if ci + 1 < NCHUNK:
                hs = start(ci + 1, (ci + 1) % NBUF)
            carry = process(slot, carry)
        flush(*carry)
        pltpu.sync_copy(acc, out_hbm.at[wid])

    return k(x0, x1, batch)


def _tc_finish(parts, uT, W1, b1, W2, b2, num_segments):
    G = num_segments

    def body(*refs):
        p_refs = refs[:len(parts)]
        u_ref, w1_ref, b1_ref, w2_ref, b2_ref, o_ref = refs[len(parts):]
        s = sum(jnp.sum(pr[...], axis=0) for pr in p_refs)
        cnt = jnp.maximum(s[2 * G:3 * G], 1.0)
        m0 = s[0:G] / cnt
        m1 = s[G:2 * G] / cnt
        u = u_ref[...]
        out4 = jnp.concatenate(
            [u[0].reshape(1, G), u[1].reshape(1, G),
             m0.reshape(1, G), m1.reshape(1, G)], axis=0)
        w1 = w1_ref[...]
        w2 = w2_ref[...]
        b1v = b1_ref[...].reshape(4, 1)
        b2v = b2_ref[...].reshape(4, 1)
        h = jnp.maximum(
            jax.lax.dot(w1, out4, preferred_element_type=jnp.float32) + b1v,
            0.0)
        y = jax.lax.dot(w2, h, preferred_element_type=jnp.float32) + b2v
        o_ref[...] = y

    return pl.pallas_call(
        body,
        out_shape=jax.ShapeDtypeStruct((4, G), jnp.float32),
    )(*parts, uT, W1, b1, W2, b2)


def kernel(x, edge_index, edge_attr, u, batch, W1, b1, W2, b2):
    n_rows = x.shape[0]
    G = u.shape[0]
    shards = [(128000, 4000), (460800, 7200), (473600, 14800),
              (537600, 8400)]
    parts = []
    off = 0
    for n_sh, chunk in shards:
        parts.append(_sc_partials(x[off:off + n_sh, 0],
                                  x[off:off + n_sh, 1],
                                  batch, off, n_sh, G, chunk))
        off += n_sh
    y4 = _tc_finish(parts, u.T, W1, b1, W2, b2, G)
    return y4.T


# speedup vs baseline: 1.0140x; 1.0140x over previous
import functools

import jax
import jax.numpy as jnp
from jax import lax
from jax.experimental import pallas as pl
from jax.experimental.pallas import tpu as pltpu
from jax.experimental.pallas import tpu_sc as plsc

_NC = 2
_NS = 16
_NW = _NC * _NS
_L = 16


def _sc_partials(x0, x1, batch, row_base, n_rows, num_segments, chunk):
    G = num_segments
    R = n_rows // _NW
    CHUNK = chunk
    NCHUNK = R // CHUNK
    BG = 5
    BLOCK = BG * _L
    NBLOCK = CHUNK // BLOCK
    NBUF = 2

    mesh = plsc.VectorSubcoreMesh(core_axis_name="c", subcore_axis_name="s")

    @functools.partial(
        pl.kernel,
        out_type=jax.ShapeDtypeStruct((_NW, 3 * G), jnp.float32),
        mesh=mesh,
        scratch_types=[
            pltpu.VMEM((CHUNK,), jnp.float32),
            pltpu.VMEM((CHUNK,), jnp.float32),
            pltpu.VMEM((CHUNK,), jnp.float32),
            pltpu.VMEM((CHUNK,), jnp.float32),
            pltpu.VMEM((CHUNK,), jnp.int32),
            pltpu.VMEM((CHUNK,), jnp.int32),
            pltpu.VMEM((3 * G,), jnp.float32),
            pltpu.SemaphoreType.DMA,
            pltpu.SemaphoreType.DMA,
        ],
        compiler_params=pltpu.CompilerParams(needs_layout_passes=False),
    )
    def k(x0_hbm, x1_hbm, b_hbm, out_hbm, x0b0, x0b1, x1b0, x1b1,
          bbf0, bbf1, acc, sem0, sem1):
        wid = lax.axis_index("s") * _NC + lax.axis_index("c")
        base = wid * R
        sems = (sem0, sem1)
        x0b = (x0b0, x0b1)
        x1b = (x1b0, x1b1)
        bbf = (bbf0, bbf1)

        zeros_f = jnp.zeros((_L,), jnp.float32)

        def zero_body(i, _):
            acc[pl.ds(i * _L, _L)] = zeros_f
            return 0

        lax.fori_loop(0, (3 * G) // _L, zero_body, 0)

        ones_f = jnp.ones((_L,), jnp.float32)
        zeros_v = jnp.zeros((_L,), jnp.float32)
        iota = lax.iota(jnp.int32, _L)
        lane3 = iota < 3
        giota = G * iota

        def flush(run_id, cnt, v0, v1):
            sa = jnp.sum(v0)
            sb = jnp.sum(v1)
            val = jnp.where(iota == 0, sa, jnp.where(iota == 1, sb, cnt))
            plsc.addupdate_scatter(acc, [run_id + giota], val, mask=lane3)

        def start(ci, slot):
            off = base + ci * CHUNK
            return (
                pltpu.async_copy(x0_hbm.at[pl.ds(off, CHUNK)],
                                 x0b[slot], sems[slot]),
                pltpu.async_copy(x1_hbm.at[pl.ds(off, CHUNK)],
                                 x1b[slot], sems[slot]),
                pltpu.async_copy(b_hbm.at[pl.ds(row_base + off, CHUNK)],
                                 bbf[slot], sems[slot]),
            )

        def group_step(slot, gb, carry):
            run_id, cnt, v0, v1 = carry
            bb = bbf[slot][pl.ds(gb, _L)]
            id_hi = bb[_L - 1]
            xa = x0b[slot][pl.ds(gb, _L)]
            xb = x1b[slot][pl.ds(gb, _L)]

            def cont(run_id, cnt, v0, v1):
                return run_id, cnt + 16.0, v0 + xa, v1 + xb

            def brk(run_id, cnt, v0, v1):
                flush(run_id, cnt, v0, v1)
                id_lo = bb[0]

                def uni(_):
                    return id_hi, 16.0, xa, xb

                def mixed(_):
                    plsc.addupdate_scatter(acc, [bb], xa)
                    plsc.addupdate_scatter(acc, [bb + G], xb)
                    plsc.addupdate_scatter(acc, [bb + 2 * G], ones_f)
                    return id_hi, 0.0, zeros_v, zeros_v

                return lax.cond(id_lo == id_hi, uni, mixed, 0)

            return lax.cond(id_hi == run_id, cont, brk, run_id, cnt, v0, v1)

        def process(slot, carry):
            def blk(i, carry):
                run_id, cnt, v0, v1 = carry
                gb = i * BLOCK
                btail = bbf[slot][pl.ds(gb + BLOCK - _L, _L)]
                id_hi = btail[_L - 1]

                def fast(run_id, cnt, v0, v1):
                    xs0 = [x0b[slot][pl.ds(gb + j * _L, _L)]
                           for j in range(BG)]
                    xs1 = [x1b[slot][pl.ds(gb + j * _L, _L)]
                           for j in range(BG)]
                    s0 = (xs0[0] + xs0[1]) + (xs0[2] + xs0[3]) + xs0[4]
                    s1 = (xs1[0] + xs1[1]) + (xs1[2] + xs1[3]) + xs1[4]
                    return run_id, cnt + float(BLOCK), v0 + s0, v1 + s1

                def slow(run_id, cnt, v0, v1):
                    carry2 = (run_id, cnt, v0, v1)
                    for j in range(BG):
                        carry2 = group_step(slot, gb + j * _L, carry2)
                    return carry2

                return lax.cond(id_hi == run_id, fast, slow,
                                run_id, cnt, v0, v1)

            return lax.fori_loop(0, NBLOCK, blk, carry)

        hs = start(0, 0)
        carry = (jnp.int32(0), jnp.float32(0.0), zeros_v, zeros_v)
        for ci in range(NCHUNK):
            slot = ci % NBUF
            for h in hs:
                h.wait()
            if ci + 1 < NCHUNK:
                hs = start(ci + 1, (ci + 1) % NBUF)
            carry = process(slot, carry)
        flush(*carry)
        pltpu.sync_copy(acc, out_hbm.at[wid])

    return k(x0, x1, batch)


def _tc_finish(parts, uT, W1, b1, W2, b2, num_segments):
    G = num_segments

    def body(*refs):
        p_refs = refs[:len(parts)]
        u_ref, w1_ref, b1_ref, w2_ref, b2_ref, o_ref = refs[len(parts):]
        s = sum(jnp.sum(pr[...], axis=0) for pr in p_refs)
        cnt = jnp.maximum(s[2 * G:3 * G], 1.0)
        m0 = s[0:G] / cnt
        m1 = s[G:2 * G] / cnt
        u = u_ref[...]
        out4 = jnp.concatenate(
            [u[0].reshape(1, G), u[1].reshape(1, G),
             m0.reshape(1, G), m1.reshape(1, G)], axis=0)
        w1 = w1_ref[...]
        w2 = w2_ref[...]
        b1v = b1_ref[...].reshape(4, 1)
        b2v = b2_ref[...].reshape(4, 1)
        h = jnp.maximum(
            jax.lax.dot(w1, out4, preferred_element_type=jnp.float32) + b1v,
            0.0)
        y = jax.lax.dot(w2, h, preferred_element_type=jnp.float32) + b2v
        o_ref[...] = y

    return pl.pallas_call(
        body,
        out_shape=jax.ShapeDtypeStruct((4, G), jnp.float32),
    )(*parts, uT, W1, b1, W2, b2)


def kernel(x, edge_index, edge_attr, u, batch, W1, b1, W2, b2):
    n_rows = x.shape[0]
    G = u.shape[0]
    shards = [(199680, 6240), (698880, 7280), (701440, 10960)]
    parts = []
    off = 0
    for n_sh, chunk in shards:
        parts.append(_sc_partials(x[off:off + n_sh, 0],
                                  x[off:off + n_sh, 1],
                                  batch, off, n_sh, G, chunk))
        off += n_sh
    y4 = _tc_finish(parts, u.T, W1, b1, W2, b2, G)
    return y4.T
